# baseline (device time: 45778 ns/iter reference)
import jax
import jax.numpy as jnp
from jax import lax
from jax.experimental import pallas as pl
from jax.experimental.pallas import tpu as pltpu

N_DEV = 4


def kernel(A, B):
    m, k_per = A.shape
    _, n = B.shape

    def body(a_ref, b_ref, out_ref, comm_ref, send_sems, recv_sems):
        my_pos = lax.axis_index("i")
        left = (my_pos - 1) % N_DEV
        right = (my_pos + 1) % N_DEV

        barrier_sem = pltpu.get_barrier_semaphore()
        for nbr in [left, right]:
            pl.semaphore_signal(
                barrier_sem, inc=1,
                device_id=(nbr,), device_id_type=pl.DeviceIdType.MESH,
            )
        pl.semaphore_wait(barrier_sem, 2)

        partial = jnp.dot(a_ref[:, :], b_ref[:, :],
                          preferred_element_type=jnp.float32)
        out_ref[:, :] = partial
        comm_ref[0, :, :] = partial

        for h in range(N_DEV - 1):
            send_slot = h % 2
            recv_slot = (h + 1) % 2
            rdma = pltpu.make_async_remote_copy(
                src_ref=comm_ref.at[send_slot],
                dst_ref=comm_ref.at[recv_slot],
                send_sem=send_sems.at[send_slot],
                recv_sem=recv_sems.at[recv_slot],
                device_id=(right,),
                device_id_type=pl.DeviceIdType.MESH,
            )
            rdma.start()
            rdma.wait()
            out_ref[:, :] = out_ref[:, :] + comm_ref[recv_slot, :, :]

        out_ref[:, :] = jnp.maximum(out_ref[:, :], 0.0)

    return pl.pallas_call(
        body,
        out_shape=jax.ShapeDtypeStruct((m, n), jnp.float32),
        in_specs=[
            pl.BlockSpec(memory_space=pltpu.VMEM),
            pl.BlockSpec(memory_space=pltpu.VMEM),
        ],
        out_specs=pl.BlockSpec(memory_space=pltpu.VMEM),
        scratch_shapes=[
            pltpu.VMEM((2, m, n), jnp.float32),
            pltpu.SemaphoreType.DMA((2,)),
            pltpu.SemaphoreType.DMA((2,)),
        ],
        compiler_params=pltpu.CompilerParams(collective_id=0),
    )(A, B)


# device time: 21653 ns/iter; 2.1142x vs baseline; 2.1142x over previous
import jax
import jax.numpy as jnp
from jax import lax
from jax.experimental import pallas as pl
from jax.experimental.pallas import tpu as pltpu

N_DEV = 4


def kernel(A, B):
    m, k_per = A.shape
    _, n = B.shape
    half = n // 4
    mhalf = m // 2

    def body(a_ref, b_ref, out_ref, rA1, rA2, rB1, rB2, send_sems, recv_sems):
        me = lax.axis_index("i")
        p1 = me ^ 1
        p2 = 3 - me

        h1 = (me ^ (me >> 1)) & 1
        h2 = me >> 1
        q1 = me >> 1
        q2 = me & 1

        k1_off = h1 * half
        g1_off = (1 - h1) * half
        k2_off = 2 * half + h2 * half
        g2_off = 2 * half + (1 - h2) * half
        myr1 = q1 * mhalf
        gr1 = (1 - q1) * mhalf
        myr2 = q2 * mhalf
        gr2 = (1 - q2) * mhalf

        barrier_sem = pltpu.get_barrier_semaphore()
        for nbr in [p1, p2]:
            pl.semaphore_signal(
                barrier_sem, inc=1,
                device_id=(nbr,), device_id_type=pl.DeviceIdType.MESH,
            )
        pl.semaphore_wait(barrier_sem, 2)

        a = a_ref[:, :]

        def mm(col_off):
            return jnp.dot(a, b_ref[:, pl.ds(col_off, half)],
                           preferred_element_type=jnp.float32)

        def xchg(sem_idx, src, dst, partner):
            return pltpu.make_async_remote_copy(
                src_ref=src, dst_ref=dst,
                send_sem=send_sems.at[sem_idx],
                recv_sem=recv_sems.at[sem_idx],
                device_id=(partner,),
                device_id_type=pl.DeviceIdType.MESH,
            )

        out_ref[:, pl.ds(g1_off, half)] = mm(g1_off)
        out_ref[:, pl.ds(g2_off, half)] = mm(g2_off)

        rdmaA1 = xchg(0, out_ref.at[:, pl.ds(g1_off, half)], rA1, p1)
        rdmaA2 = xchg(1, out_ref.at[:, pl.ds(g2_off, half)], rA2, p2)
        rdmaA1.start()
        rdmaA2.start()

        out_ref[:, pl.ds(k1_off, half)] = mm(k1_off)
        out_ref[:, pl.ds(k2_off, half)] = mm(k2_off)

        rdmaA1.wait()
        rdmaA2.wait()
        out_ref[:, pl.ds(k1_off, half)] = out_ref[:, pl.ds(k1_off, half)] + rA1[:, :]
        out_ref[:, pl.ds(k2_off, half)] = out_ref[:, pl.ds(k2_off, half)] + rA2[:, :]

        rdmaB1 = xchg(2, out_ref.at[pl.ds(gr1, mhalf), pl.ds(k1_off, half)],
                      rB1, p2)
        rdmaB2 = xchg(3, out_ref.at[pl.ds(gr2, mhalf), pl.ds(k2_off, half)],
                      rB2, p1)
        rdmaB1.start()
        rdmaB2.start()
        rdmaB1.wait()
        rdmaB2.wait()
        out_ref[pl.ds(myr1, mhalf), pl.ds(k1_off, half)] = jnp.maximum(
            out_ref[pl.ds(myr1, mhalf), pl.ds(k1_off, half)] + rB1[:, :], 0.0)
        out_ref[pl.ds(myr2, mhalf), pl.ds(k2_off, half)] = jnp.maximum(
            out_ref[pl.ds(myr2, mhalf), pl.ds(k2_off, half)] + rB2[:, :], 0.0)

        rdmaC1 = xchg(4, out_ref.at[pl.ds(myr1, mhalf), pl.ds(k1_off, half)],
                      out_ref.at[pl.ds(myr1, mhalf), pl.ds(k1_off, half)], p2)
        rdmaC2 = xchg(5, out_ref.at[pl.ds(myr2, mhalf), pl.ds(k2_off, half)],
                      out_ref.at[pl.ds(myr2, mhalf), pl.ds(k2_off, half)], p1)
        rdmaC1.start()
        rdmaC2.start()
        rdmaC1.wait()
        rdmaC2.wait()

        rdmaD1 = xchg(6, out_ref.at[:, pl.ds(k1_off, half)],
                      out_ref.at[:, pl.ds(k1_off, half)], p1)
        rdmaD2 = xchg(7, out_ref.at[:, pl.ds(k2_off, half)],
                      out_ref.at[:, pl.ds(k2_off, half)], p2)
        rdmaD1.start()
        rdmaD2.start()
        rdmaD1.wait()
        rdmaD2.wait()

    return pl.pallas_call(
        body,
        out_shape=jax.ShapeDtypeStruct((m, n), jnp.float32),
        in_specs=[
            pl.BlockSpec(memory_space=pltpu.VMEM),
            pl.BlockSpec(memory_space=pltpu.VMEM),
        ],
        out_specs=pl.BlockSpec(memory_space=pltpu.VMEM),
        scratch_shapes=[
            pltpu.VMEM((m, n // 4), jnp.float32),
            pltpu.VMEM((m, n // 4), jnp.float32),
            pltpu.VMEM((m // 2, n // 4), jnp.float32),
            pltpu.VMEM((m // 2, n // 4), jnp.float32),
            pltpu.SemaphoreType.DMA((8,)),
            pltpu.SemaphoreType.DMA((8,)),
        ],
        compiler_params=pltpu.CompilerParams(collective_id=0),
    )(A, B)


# device time: 17749 ns/iter; 2.5792x vs baseline; 1.2200x over previous
import jax
import jax.numpy as jnp
from jax import lax
from jax.experimental import pallas as pl
from jax.experimental.pallas import tpu as pltpu

N_DEV = 4


def kernel(A, B):
    m, k_per = A.shape
    _, n = B.shape
    blk = n // N_DEV

    def body(a_ref, b_ref, out_ref,
             sb1, sb2, sbd, r1, r2, rd,
             agb, agr1, agr2, agrd,
             send_sems, recv_sems):
        me = lax.axis_index("i")
        p1 = me ^ 1
        p2 = 3 - me
        dg = me ^ 2

        barrier_sem = pltpu.get_barrier_semaphore()
        for nbr in [p1, p2, dg]:
            pl.semaphore_signal(
                barrier_sem, inc=1,
                device_id=(nbr,), device_id_type=pl.DeviceIdType.MESH,
            )
        pl.semaphore_wait(barrier_sem, 3)

        a = a_ref[:, :]

        def mm(col_off):
            return jnp.dot(a, b_ref[:, pl.ds(col_off, blk)],
                           preferred_element_type=jnp.float32)

        def xchg(sem_idx, src, dst, partner):
            return pltpu.make_async_remote_copy(
                src_ref=src, dst_ref=dst,
                send_sem=send_sems.at[sem_idx],
                recv_sem=recv_sems.at[sem_idx],
                device_id=(partner,),
                device_id_type=pl.DeviceIdType.MESH,
            )

        sb1[:, :] = mm(p1 * blk).astype(jnp.bfloat16)
        rs1 = xchg(0, sb1, r1, p1)
        rs1.start()
        sb2[:, :] = mm(p2 * blk).astype(jnp.bfloat16)
        rs2 = xchg(1, sb2, r2, p2)
        rs2.start()
        sbd[:, :] = mm(dg * blk).astype(jnp.bfloat16)
        rsd = xchg(2, sbd, rd, dg)
        rsd.start()

        own = mm(me * blk)

        rs1.wait()
        rs2.wait()
        rsd.wait()
        mine = jnp.maximum(
            own
            + r1[:, :].astype(jnp.float32)
            + r2[:, :].astype(jnp.float32)
            + rd[:, :].astype(jnp.float32),
            0.0,
        )
        out_ref[:, pl.ds(me * blk, blk)] = mine
        agb[:, :] = mine.astype(jnp.bfloat16)

        ag1 = xchg(3, agb, agr1, p1)
        ag2 = xchg(4, agb, agr2, p2)
        agd = xchg(5, agb, agrd, dg)
        ag1.start()
        ag2.start()
        agd.start()
        ag1.wait()
        ag2.wait()
        agd.wait()
        out_ref[:, pl.ds(p1 * blk, blk)] = agr1[:, :].astype(jnp.float32)
        out_ref[:, pl.ds(p2 * blk, blk)] = agr2[:, :].astype(jnp.float32)
        out_ref[:, pl.ds(dg * blk, blk)] = agrd[:, :].astype(jnp.float32)

    comm = lambda: pltpu.VMEM((m, blk), jnp.bfloat16)
    return pl.pallas_call(
        body,
        out_shape=jax.ShapeDtypeStruct((m, n), jnp.float32),
        in_specs=[
            pl.BlockSpec(memory_space=pltpu.VMEM),
            pl.BlockSpec(memory_space=pltpu.VMEM),
        ],
        out_specs=pl.BlockSpec(memory_space=pltpu.VMEM),
        scratch_shapes=[
            comm(), comm(), comm(),
            comm(), comm(), comm(),
            comm(),
            comm(), comm(), comm(),
            pltpu.SemaphoreType.DMA((6,)),
            pltpu.SemaphoreType.DMA((6,)),
        ],
        compiler_params=pltpu.CompilerParams(collective_id=0),
    )(A, B)


# device time: 16324 ns/iter; 2.8043x vs baseline; 1.0873x over previous
import jax
import jax.numpy as jnp
from jax import lax
from jax.experimental import pallas as pl
from jax.experimental.pallas import tpu as pltpu

N_DEV = 4


def kernel(A, B):
    m, k_per = A.shape
    _, n = B.shape
    blk = n // N_DEV

    def body(a_ref, b_ref, out_ref,
             sb1, sb2, sbd, r1, r2, rd,
             agb, agr1, agr2, agrd,
             send_sems, recv_sems):
        me = lax.axis_index("i")
        p1 = me ^ 1
        p2 = 3 - me
        dg = me ^ 2

        barrier_sem = pltpu.get_barrier_semaphore()
        for nbr in [p1, p2, dg]:
            pl.semaphore_signal(
                barrier_sem, inc=1,
                device_id=(nbr,), device_id_type=pl.DeviceIdType.MESH,
            )

        a = a_ref[:, :].astype(jnp.bfloat16)

        def mm(col_off):
            return jnp.dot(a, b_ref[:, pl.ds(col_off, blk)].astype(jnp.bfloat16),
                           preferred_element_type=jnp.float32)

        def xchg(sem_idx, src, dst, partner):
            return pltpu.make_async_remote_copy(
                src_ref=src, dst_ref=dst,
                send_sem=send_sems.at[sem_idx],
                recv_sem=recv_sems.at[sem_idx],
                device_id=(partner,),
                device_id_type=pl.DeviceIdType.MESH,
            )

        sbd[:, :] = mm(dg * blk).astype(jnp.bfloat16)
        pl.semaphore_wait(barrier_sem, 3)
        rsd = xchg(2, sbd, rd, dg)
        rsd.start()
        sb1[:, :] = mm(p1 * blk).astype(jnp.bfloat16)
        rs1 = xchg(0, sb1, r1, p1)
        rs1.start()
        sb2[:, :] = mm(p2 * blk).astype(jnp.bfloat16)
        rs2 = xchg(1, sb2, r2, p2)
        rs2.start()

        own = mm(me * blk)

        rs1.wait()
        acc = own + r1[:, :].astype(jnp.float32)
        rs2.wait()
        acc = acc + r2[:, :].astype(jnp.float32)
        rsd.wait()
        mine = jnp.maximum(acc + rd[:, :].astype(jnp.float32), 0.0)
        out_ref[:, pl.ds(me * blk, blk)] = mine
        agb[:, :] = mine.astype(jnp.bfloat16)

        agd = xchg(5, agb, agrd, dg)
        ag1 = xchg(3, agb, agr1, p1)
        ag2 = xchg(4, agb, agr2, p2)
        agd.start()
        ag1.start()
        ag2.start()
        ag1.wait()
        out_ref[:, pl.ds(p1 * blk, blk)] = agr1[:, :].astype(jnp.float32)
        ag2.wait()
        out_ref[:, pl.ds(p2 * blk, blk)] = agr2[:, :].astype(jnp.float32)
        agd.wait()
        out_ref[:, pl.ds(dg * blk, blk)] = agrd[:, :].astype(jnp.float32)

    comm = lambda: pltpu.VMEM((m, blk), jnp.bfloat16)
    return pl.pallas_call(
        body,
        out_shape=jax.ShapeDtypeStruct((m, n), jnp.float32),
        in_specs=[
            pl.BlockSpec(memory_space=pltpu.VMEM),
            pl.BlockSpec(memory_space=pltpu.VMEM),
        ],
        out_specs=pl.BlockSpec(memory_space=pltpu.VMEM),
        scratch_shapes=[
            comm(), comm(), comm(),
            comm(), comm(), comm(),
            comm(),
            comm(), comm(), comm(),
            pltpu.SemaphoreType.DMA((6,)),
            pltpu.SemaphoreType.DMA((6,)),
        ],
        compiler_params=pltpu.CompilerParams(collective_id=0),
    )(A, B)


# device time: 14841 ns/iter; 3.0846x vs baseline; 1.0999x over previous
import jax
import jax.numpy as jnp
from jax import lax
from jax.experimental import pallas as pl
from jax.experimental.pallas import tpu as pltpu

N_DEV = 4


def kernel(A, B):
    m, k_per = A.shape
    _, n = B.shape
    blk = n // N_DEV
    mh = m // 2

    def body(a_ref, b_ref, out_ref,
             sb1, sb2, sbd, r1, r2, rd,
             agb, agr1, agr2, agrd,
             send_sems, recv_sems):
        me = lax.axis_index("i")
        p1 = me ^ 1
        p2 = 3 - me
        dg = me ^ 2

        barrier_sem = pltpu.get_barrier_semaphore()
        for nbr in [p1, p2, dg]:
            pl.semaphore_signal(
                barrier_sem, inc=1,
                device_id=(nbr,), device_id_type=pl.DeviceIdType.MESH,
            )

        a = a_ref[:, :].astype(jnp.bfloat16)

        def mm(row, col_off):
            return jnp.dot(
                a[row * mh:(row + 1) * mh, :],
                b_ref[:, pl.ds(col_off, blk)].astype(jnp.bfloat16),
                preferred_element_type=jnp.float32,
            )

        def xchg(sem_idx, src, dst, partner):
            return pltpu.make_async_remote_copy(
                src_ref=src, dst_ref=dst,
                send_sem=send_sems.at[sem_idx],
                recv_sem=recv_sems.at[sem_idx],
                device_id=(partner,),
                device_id_type=pl.DeviceIdType.MESH,
            )

        def rs_send(sem_idx, buf, rbuf, partner, col_off, row):
            buf[pl.ds(row * mh, mh), :] = mm(row, col_off).astype(jnp.bfloat16)
            op = xchg(sem_idx, buf.at[pl.ds(row * mh, mh)],
                      rbuf.at[pl.ds(row * mh, mh)], partner)
            op.start()
            return op

        sbd[pl.ds(0, mh), :] = mm(0, dg * blk).astype(jnp.bfloat16)
        pl.semaphore_wait(barrier_sem, 3)
        rsd_t = xchg(0, sbd.at[pl.ds(0, mh)], rd.at[pl.ds(0, mh)], dg)
        rsd_t.start()
        rs1_t = rs_send(1, sb1, r1, p1, p1 * blk, 0)
        rs2_t = rs_send(2, sb2, r2, p2, p2 * blk, 0)
        rsd_b = rs_send(3, sbd, rd, dg, dg * blk, 1)
        rs1_b = rs_send(4, sb1, r1, p1, p1 * blk, 1)
        rs2_b = rs_send(5, sb2, r2, p2, p2 * blk, 1)

        def reduce_half(row, own, ops):
            o1, o2, od = ops
            o1.wait()
            acc = own + r1[pl.ds(row * mh, mh), :].astype(jnp.float32)
            o2.wait()
            acc = acc + r2[pl.ds(row * mh, mh), :].astype(jnp.float32)
            od.wait()
            mine = jnp.maximum(
                acc + rd[pl.ds(row * mh, mh), :].astype(jnp.float32), 0.0)
            agb[pl.ds(row * mh, mh), :] = mine.astype(jnp.bfloat16)
            ops_ag = []
            for sem_idx, rbuf, partner in (
                (6 + 3 * row, agrd, dg),
                (7 + 3 * row, agr1, p1),
                (8 + 3 * row, agr2, p2),
            ):
                op = xchg(sem_idx, agb.at[pl.ds(row * mh, mh)],
                          rbuf.at[pl.ds(row * mh, mh)], partner)
                op.start()
                ops_ag.append(op)
            out_ref[pl.ds(row * mh, mh), pl.ds(me * blk, blk)] = mine
            return ops_ag

        own_t = mm(0, me * blk)
        ag_t = reduce_half(0, own_t, (rs1_t, rs2_t, rsd_t))

        own_b = mm(1, me * blk)
        ag_b = reduce_half(1, own_b, (rs1_b, rs2_b, rsd_b))

        for row, ops in ((0, ag_t), (1, ag_b)):
            opd, op1, op2 = ops
            rows = pl.ds(row * mh, mh)
            op1.wait()
            out_ref[rows, pl.ds(p1 * blk, blk)] = (
                agr1[rows, :].astype(jnp.float32))
            op2.wait()
            out_ref[rows, pl.ds(p2 * blk, blk)] = (
                agr2[rows, :].astype(jnp.float32))
            opd.wait()
            out_ref[rows, pl.ds(dg * blk, blk)] = (
                agrd[rows, :].astype(jnp.float32))

    comm = lambda: pltpu.VMEM((m, blk), jnp.bfloat16)
    return pl.pallas_call(
        body,
        out_shape=jax.ShapeDtypeStruct((m, n), jnp.float32),
        in_specs=[
            pl.BlockSpec(memory_space=pltpu.VMEM),
            pl.BlockSpec(memory_space=pltpu.VMEM),
        ],
        out_specs=pl.BlockSpec(memory_space=pltpu.VMEM),
        scratch_shapes=[
            comm(), comm(), comm(),
            comm(), comm(), comm(),
            comm(),
            comm(), comm(), comm(),
            pltpu.SemaphoreType.DMA((12,)),
            pltpu.SemaphoreType.DMA((12,)),
        ],
        compiler_params=pltpu.CompilerParams(collective_id=0),
    )(A, B)


# device time: 14322 ns/iter; 3.1963x vs baseline; 1.0362x over previous
import jax
import jax.numpy as jnp
from jax import lax
from jax.experimental import pallas as pl
from jax.experimental.pallas import tpu as pltpu

N_DEV = 4
R = 4


def kernel(A, B):
    m, k_per = A.shape
    _, n = B.shape
    blk = n // N_DEV
    mq = m // R

    def body(a_ref, b_ref, out_ref,
             sb1, sb2, sbd, r1, r2, rd,
             agb, agr1, agr2, agrd,
             send_sems, recv_sems):
        me = lax.axis_index("i")
        p1 = me ^ 1
        p2 = 3 - me
        dg = me ^ 2

        barrier_sem = pltpu.get_barrier_semaphore()
        for nbr in [p1, p2, dg]:
            pl.semaphore_signal(
                barrier_sem, inc=1,
                device_id=(nbr,), device_id_type=pl.DeviceIdType.MESH,
            )

        a = a_ref[:, :].astype(jnp.bfloat16)

        def mm(q, col_off):
            return jnp.dot(
                a[q * mq:(q + 1) * mq, :],
                b_ref[:, pl.ds(col_off, blk)].astype(jnp.bfloat16),
                preferred_element_type=jnp.float32,
            )

        def xchg(sem_idx, src, dst, partner):
            return pltpu.make_async_remote_copy(
                src_ref=src, dst_ref=dst,
                send_sem=send_sems.at[sem_idx],
                recv_sem=recv_sems.at[sem_idx],
                device_id=(partner,),
                device_id_type=pl.DeviceIdType.MESH,
            )

        first = True
        rs_ops = []
        for q in range(R):
            rows = pl.ds(q * mq, mq)
            ops = []
            for rel, (buf, rbuf, partner) in enumerate(
                ((sbd, rd, dg), (sb1, r1, p1), (sb2, r2, p2))
            ):
                col_off = partner * blk
                buf[rows, :] = mm(q, col_off).astype(jnp.bfloat16)
                if first:
                    pl.semaphore_wait(barrier_sem, 3)
                    first = False
                op = xchg(3 * q + rel, buf.at[rows], rbuf.at[rows], partner)
                op.start()
                ops.append(op)
            rs_ops.append(ops)

        ag_ops = []
        for q in range(R):
            rows = pl.ds(q * mq, mq)
            od, o1, o2 = rs_ops[q]
            acc = mm(q, me * blk)
            o1.wait()
            acc = acc + r1[rows, :].astype(jnp.float32)
            o2.wait()
            acc = acc + r2[rows, :].astype(jnp.float32)
            od.wait()
            mine = jnp.maximum(acc + rd[rows, :].astype(jnp.float32), 0.0)
            agb[rows, :] = mine.astype(jnp.bfloat16)
            ops = []
            for rel, (rbuf, partner) in enumerate(
                ((agrd, dg), (agr1, p1), (agr2, p2))
            ):
                op = xchg(3 * R + 3 * q + rel,
                          agb.at[rows], rbuf.at[rows], partner)
                op.start()
                ops.append(op)
            ag_ops.append(ops)
            out_ref[rows, pl.ds(me * blk, blk)] = mine

        for q in range(R):
            rows = pl.ds(q * mq, mq)
            opd, op1, op2 = ag_ops[q]
            op1.wait()
            out_ref[rows, pl.ds(p1 * blk, blk)] = (
                agr1[rows, :].astype(jnp.float32))
            op2.wait()
            out_ref[rows, pl.ds(p2 * blk, blk)] = (
                agr2[rows, :].astype(jnp.float32))
            opd.wait()
            out_ref[rows, pl.ds(dg * blk, blk)] = (
                agrd[rows, :].astype(jnp.float32))

    comm = lambda: pltpu.VMEM((m, blk), jnp.bfloat16)
    return pl.pallas_call(
        body,
        out_shape=jax.ShapeDtypeStruct((m, n), jnp.float32),
        in_specs=[
            pl.BlockSpec(memory_space=pltpu.VMEM),
            pl.BlockSpec(memory_space=pltpu.VMEM),
        ],
        out_specs=pl.BlockSpec(memory_space=pltpu.VMEM),
        scratch_shapes=[
            comm(), comm(), comm(),
            comm(), comm(), comm(),
            comm(),
            comm(), comm(), comm(),
            pltpu.SemaphoreType.DMA((6 * R,)),
            pltpu.SemaphoreType.DMA((6 * R,)),
        ],
        compiler_params=pltpu.CompilerParams(collective_id=0),
    )(A, B)
